# baseline (speedup 1.0000x reference)
import jax
import jax.numpy as jnp
from jax import lax
from jax.experimental import pallas as pl
from jax.experimental.pallas import tpu as pltpu
from jax.experimental.pallas import tpu_sc as plsc

_N = 20000
_G = 100
_C = 80
_P = 512
_LANES = 16
_CHUNK = 2000
_FIFTH_ROOT_MAGIC = 851980270


_KSTEPS = _N // _CHUNK


def _tc_gather_body(pos_ref, cls_ref, sig_ref, acc_ref):
    k = pl.program_id(0)
    ids = jax.lax.broadcasted_iota(jnp.int32, (_P, _CHUNK), 1) + k * _CHUNK
    onehot = jnp.where(pos_ref[...] == ids, 1.0, 0.0).astype(jnp.bfloat16)
    part = jax.lax.dot_general(
        onehot, cls_ref[0].astype(jnp.bfloat16),
        dimension_numbers=(((1,), (0,)), ((), ())),
        preferred_element_type=jnp.float32)

    @pl.when(k == 0)
    def _init():
        acc_ref[...] = part

    @pl.when(k > 0)
    def _acc():
        acc_ref[...] += part

    @pl.when(k == _KSTEPS - 1)
    def _fin():
        sig_ref[...] = 1.0 / (1.0 + jnp.exp(-acc_ref[...]))


def _tc_gather(pos_2d, cls_scores):
    cls3 = cls_scores.reshape(_KSTEPS, _CHUNK, _C)
    return pl.pallas_call(
        _tc_gather_body,
        grid=(_KSTEPS,),
        in_specs=[
            pl.BlockSpec((_P, 1), lambda k: (0, 0)),
            pl.BlockSpec((1, _CHUNK, _C), lambda k: (k, 0, 0)),
        ],
        out_specs=pl.BlockSpec((_P, _C), lambda k: (0, 0)),
        out_shape=jax.ShapeDtypeStruct((_P, _C), jnp.float32),
        scratch_shapes=[pltpu.VMEM((_P, _C), jnp.float32)],
    )(pos_2d, cls3)


def _sc_body(pos_hbm, ptf_hbm, sig_hbm, gtbf_hbm, gtl_hbm, out_hbm,
             idx_v, coord_v, csc_v, gtb_v, gtl_v, qbuf, sem_a, sem_b):
    c = lax.axis_index("c")
    s = lax.axis_index("s")
    wid = s * 2 + c

    pltpu.sync_copy(pos_hbm.at[pl.ds(wid * _LANES, _LANES)], idx_v)
    idx = idx_v[...]

    coord_copies = [
        pltpu.async_copy(ptf_hbm.at[idx + cc * _N], coord_v.at[cc], sem_a)
        for cc in range(4)
    ]
    cls_copy = pltpu.async_copy(sig_hbm.at[pl.ds(wid * _LANES, _LANES)],
                                csc_v, sem_b)

    pltpu.sync_copy(gtbf_hbm, gtb_v)
    pltpu.sync_copy(gtl_hbm, gtl_v)
    for cp in coord_copies:
        cp.wait()
    cls_copy.wait()

    lane = lax.iota(jnp.int32, _LANES)
    px1 = coord_v[0]
    py1 = coord_v[1]
    px2 = coord_v[2]
    py2 = coord_v[3]
    area1 = (px2 - px1) * (py2 - py1)

    def _gt_step(g, m):
        g4 = jnp.full((_LANES,), g * 4, jnp.int32)
        gx1 = plsc.load_gather(gtb_v, [g4])
        gy1 = plsc.load_gather(gtb_v, [g4 + 1])
        gx2 = plsc.load_gather(gtb_v, [g4 + 2])
        gy2 = plsc.load_gather(gtb_v, [g4 + 3])
        w = jnp.maximum(jnp.minimum(px2, gx2) - jnp.maximum(px1, gx1), 0.0)
        h = jnp.maximum(jnp.minimum(py2, gy2) - jnp.maximum(py1, gy1), 0.0)
        inter = w * h
        area2 = (gx2 - gx1) * (gy2 - gy1)
        union = jnp.maximum(area1 + area2 - inter, 1e-6)
        iou = inter / union
        lab = plsc.load_gather(gtl_v, [jnp.full((_LANES,), g, jnp.int32)])
        cls = plsc.load_gather(csc_v, [lane, lab])
        iou2 = iou * iou
        return jnp.maximum(m, iou2 * iou2 * cls)

    m = lax.fori_loop(0, _G, _gt_step, jnp.zeros((_LANES,), jnp.float32),
                      unroll=4)

    bits = plsc.bitcast(m, jnp.int32)
    seed_bits = (bits.astype(jnp.float32) * 0.2).astype(jnp.int32)
    y = plsc.bitcast(seed_bits + _FIFTH_ROOT_MAGIC, jnp.float32)
    for _ in range(3):
        y2 = y * y
        y4 = y2 * y2
        y = 0.8 * y + 0.2 * m / y4
    qbuf[...] = jnp.where(m > 0.0, y, 0.0)
    pltpu.async_copy(qbuf, out_hbm.at[idx_v], sem_a).wait()


@jax.jit
def _run(pos_i32, ptflat, cls_scores, gtbf, gtl_i32):
    sig512 = _tc_gather(pos_i32.reshape(_P, 1), cls_scores)

    mesh = plsc.VectorSubcoreMesh(core_axis_name="c", subcore_axis_name="s")
    f = pl.kernel(
        _sc_body,
        out_type=(),
        mesh=mesh,
        compiler_params=pltpu.CompilerParams(
            needs_layout_passes=False, use_tc_tiling_on_sc=False),
        scratch_types=[
            pltpu.VMEM((_LANES,), jnp.int32),
            pltpu.VMEM((4, _LANES), jnp.float32),
            pltpu.VMEM((_LANES, _C), jnp.float32),
            pltpu.VMEM((4 * _G,), jnp.float32),
            pltpu.VMEM((_G,), jnp.int32),
            pltpu.VMEM((_LANES,), jnp.float32),
            pltpu.SemaphoreType.DMA,
            pltpu.SemaphoreType.DMA,
        ],
    )
    out_ref = jax.new_ref(jnp.zeros((_N,), jnp.float32))
    f(pos_i32, ptflat, sig512, gtbf, gtl_i32, out_ref)
    return out_ref[...]


def kernel(bboxes, cls_scores, bbox_preds, gt_bboxes, bbox_levels, positive_inds, gt_labels):
    del bboxes, bbox_levels
    pos_i32 = positive_inds.astype(jnp.int32)
    gtl_i32 = gt_labels.astype(jnp.int32)
    gtbf = gt_bboxes[:, :4].reshape(-1)
    ptflat = bbox_preds.T.reshape(-1)
    return _run(pos_i32, ptflat, cls_scores, gtbf, gtl_i32)


# speedup vs baseline: 1.4484x; 1.4484x over previous
import jax
import jax.numpy as jnp
from jax import lax
from jax.experimental import pallas as pl
from jax.experimental.pallas import tpu as pltpu
from jax.experimental.pallas import tpu_sc as plsc

_N = 20000
_G = 100
_C = 80
_P = 512
_LANES = 16
_CHUNK = 2000
_FIFTH_ROOT_MAGIC = 851980270


_KSTEPS = _N // _CHUNK


def _tc_gather_body(pos_ref, cls_ref, sig_ref, acc_ref):
    k = pl.program_id(0)
    ids = jax.lax.broadcasted_iota(jnp.int32, (_P, _CHUNK), 1) + k * _CHUNK
    onehot = jnp.where(pos_ref[...] == ids, 1.0, 0.0).astype(jnp.bfloat16)
    part = jax.lax.dot_general(
        onehot, cls_ref[...].astype(jnp.bfloat16),
        dimension_numbers=(((1,), (0,)), ((), ())),
        preferred_element_type=jnp.float32)

    @pl.when(k == 0)
    def _init():
        acc_ref[...] = part

    @pl.when(k > 0)
    def _acc():
        acc_ref[...] += part

    @pl.when(k == _KSTEPS - 1)
    def _fin():
        sig_ref[...] = 1.0 / (1.0 + jnp.exp(-acc_ref[...]))


def _tc_gather(pos_2d, cls_scores):
    return pl.pallas_call(
        _tc_gather_body,
        grid=(_KSTEPS,),
        in_specs=[
            pl.BlockSpec((_P, 1), lambda k: (0, 0)),
            pl.BlockSpec((_CHUNK, _C), lambda k: (k, 0)),
        ],
        out_specs=pl.BlockSpec((_P, _C), lambda k: (0, 0)),
        out_shape=jax.ShapeDtypeStruct((_P, _C), jnp.float32),
        scratch_shapes=[pltpu.VMEM((_P, _C), jnp.float32)],
    )(pos_2d, cls_scores)


def _sc_body(pos_hbm, ptf_hbm, sig_hbm, gtbf_hbm, gtl_hbm, out_hbm,
             idx_v, coord_v, csc_v, gtb_v, gtl_v, qbuf, sem_a, sem_b):
    c = lax.axis_index("c")
    s = lax.axis_index("s")
    wid = s * 2 + c

    pltpu.sync_copy(pos_hbm.at[pl.ds(wid * _LANES, _LANES)], idx_v)
    idx = idx_v[...]

    coord_copies = [
        pltpu.async_copy(ptf_hbm.at[idx + cc * _N], coord_v.at[cc], sem_a)
        for cc in range(4)
    ]
    cls_copy = pltpu.async_copy(sig_hbm.at[pl.ds(wid * _LANES, _LANES)],
                                csc_v, sem_b)

    pltpu.sync_copy(gtbf_hbm, gtb_v)
    pltpu.sync_copy(gtl_hbm, gtl_v)
    for cp in coord_copies:
        cp.wait()
    cls_copy.wait()

    lane = lax.iota(jnp.int32, _LANES)
    px1 = coord_v[0]
    py1 = coord_v[1]
    px2 = coord_v[2]
    py2 = coord_v[3]
    area1 = (px2 - px1) * (py2 - py1)

    def _gt_step(g, m):
        g4 = jnp.full((_LANES,), g * 4, jnp.int32)
        gx1 = plsc.load_gather(gtb_v, [g4])
        gy1 = plsc.load_gather(gtb_v, [g4 + 1])
        gx2 = plsc.load_gather(gtb_v, [g4 + 2])
        gy2 = plsc.load_gather(gtb_v, [g4 + 3])
        w = jnp.maximum(jnp.minimum(px2, gx2) - jnp.maximum(px1, gx1), 0.0)
        h = jnp.maximum(jnp.minimum(py2, gy2) - jnp.maximum(py1, gy1), 0.0)
        inter = w * h
        area2 = (gx2 - gx1) * (gy2 - gy1)
        union = jnp.maximum(area1 + area2 - inter, 1e-6)
        iou = inter / union
        lab = plsc.load_gather(gtl_v, [jnp.full((_LANES,), g, jnp.int32)])
        cls = plsc.load_gather(csc_v, [lane, lab])
        iou2 = iou * iou
        return jnp.maximum(m, iou2 * iou2 * cls)

    m = lax.fori_loop(0, _G, _gt_step, jnp.zeros((_LANES,), jnp.float32),
                      unroll=4)

    bits = plsc.bitcast(m, jnp.int32)
    seed_bits = (bits.astype(jnp.float32) * 0.2).astype(jnp.int32)
    y = plsc.bitcast(seed_bits + _FIFTH_ROOT_MAGIC, jnp.float32)
    for _ in range(3):
        y2 = y * y
        y4 = y2 * y2
        y = 0.8 * y + 0.2 * m / y4
    qbuf[...] = jnp.where(m > 0.0, y, 0.0)
    pltpu.async_copy(qbuf, out_hbm.at[idx_v], sem_a).wait()


@jax.jit
def _run(pos_i32, ptflat, cls_scores, gtbf, gtl_i32):
    sig512 = _tc_gather(pos_i32.reshape(_P, 1), cls_scores)

    mesh = plsc.VectorSubcoreMesh(core_axis_name="c", subcore_axis_name="s")
    f = pl.kernel(
        _sc_body,
        out_type=(),
        mesh=mesh,
        compiler_params=pltpu.CompilerParams(
            needs_layout_passes=False, use_tc_tiling_on_sc=False),
        scratch_types=[
            pltpu.VMEM((_LANES,), jnp.int32),
            pltpu.VMEM((4, _LANES), jnp.float32),
            pltpu.VMEM((_LANES, _C), jnp.float32),
            pltpu.VMEM((4 * _G,), jnp.float32),
            pltpu.VMEM((_G,), jnp.int32),
            pltpu.VMEM((_LANES,), jnp.float32),
            pltpu.SemaphoreType.DMA,
            pltpu.SemaphoreType.DMA,
        ],
    )
    out_ref = jax.new_ref(jnp.zeros((_N,), jnp.float32))
    f(pos_i32, ptflat, sig512, gtbf, gtl_i32, out_ref)
    return out_ref[...]


def kernel(bboxes, cls_scores, bbox_preds, gt_bboxes, bbox_levels, positive_inds, gt_labels):
    del bboxes, bbox_levels
    pos_i32 = positive_inds.astype(jnp.int32)
    gtl_i32 = gt_labels.astype(jnp.int32)
    gtbf = gt_bboxes[:, :4].reshape(-1)
    ptflat = bbox_preds.T.reshape(-1)
    return _run(pos_i32, ptflat, cls_scores, gtbf, gtl_i32)
